# baseline (speedup 1.0000x reference)
import functools

import jax
import jax.numpy as jnp
from jax import lax
from jax.experimental import pallas as pl
from jax.experimental.pallas import tpu as pltpu
from jax.experimental.pallas import tpu_sc as plsc

N = 10000
H = 128
E = 320000
ETYPES = 5
_ATT_BIAS = (-4.0, -4.0, -4.0, -4.0, -2.772)

NC, NS, L = 2, 16, 16
FH = H // NC
K = 128
NBUF = 3
U = 2 * NBUF
CPT = 162
EPAD = NS * CPT * K
NPAD = 10240
RPT = NPAD // NS
DUMMY = N


def _sc_agg(xh, idx):
    mesh = plsc.VectorSubcoreMesh(core_axis_name="c", subcore_axis_name="s")

    @functools.partial(
        pl.kernel,
        out_type=(
            jax.ShapeDtypeStruct((NC, ETYPES, NPAD, FH), jnp.float32),
            jax.ShapeDtypeStruct((ETYPES, NS, NPAD), jnp.float32),
        ),
        mesh=mesh,
        scratch_types=[
            pltpu.VMEM((U, 2, K), jnp.int32),
            pltpu.VMEM((NBUF, K, FH), jnp.float32),
            pltpu.VMEM((NPAD,), jnp.float32),
            pltpu.VMEM_SHARED((NPAD, FH), jnp.float32),
            pltpu.VMEM_SHARED((NPAD, FH), jnp.float32),
            pltpu.SemaphoreType.DMA((U,)),
            pltpu.SemaphoreType.DMA((NBUF,)),
        ],
        compiler_params=pltpu.CompilerParams(use_tc_tiling_on_sc=False,
                                             needs_layout_passes=False),
    )
    def k(xh_hbm, idx_hbm, out_hbm, cnt_hbm, idx_v, rows_v, hist_v, x_sh,
          sums_sh, isem, gsem):
        c = lax.axis_index("c")
        s = lax.axis_index("s")
        zvec = jnp.zeros((L,), jnp.float32)
        ones = jnp.ones((L,), jnp.float32)

        def zrow(i, carry):
            for j in range(FH // L):
                rows_v[0, i, pl.ds(j * L, L)] = zvec
            return carry

        def zero_own_slice():
            lax.fori_loop(0, K, zrow, 0)
            for q in range(RPT // K):
                pltpu.sync_copy(rows_v.at[0],
                                sums_sh.at[pl.ds(s * RPT + q * K, K)])

        def zero_hist():
            def zh(i, carry):
                hist_v[pl.ds(i * L, L)] = zvec
                return carry
            lax.fori_loop(0, NPAD // L, zh, 0)

        def load_idx(e, chunk, q):
            pltpu.async_copy(idx_hbm.at[e, s, chunk], idx_v.at[q],
                             isem.at[q])

        def wait_idx(q):
            pltpu.make_async_copy(idx_hbm.at[0, 0, 0], idx_v.at[q],
                                  isem.at[q]).wait()

        def start_gather(q, b):
            pltpu.async_copy(x_sh.at[idx_v.at[q, 0]], rows_v.at[b],
                             gsem.at[b])

        def wait_gather(b):
            pltpu.make_async_copy(xh_hbm.at[0, pl.ds(0, K)], rows_v.at[b],
                                  gsem.at[b]).wait()

        def scatter(q, b):
            pltpu.sync_copy(rows_v.at[b], sums_sh.at[idx_v.at[q, 1]],
                            add=True)

        def count(q):
            @pl.when(c == 0)
            def _():
                for j in range(K // L):
                    dstv = idx_v[q, 1, pl.ds(j * L, L)]
                    plsc.addupdate_scatter(hist_v, [dstv], ones)

        pltpu.sync_copy(xh_hbm.at[c, pl.ds(s * RPT, RPT)],
                        x_sh.at[pl.ds(s * RPT, RPT)])
        zero_hist()
        zero_own_slice()

        for e in range(ETYPES):
            plsc.subcore_barrier()
            for q in range(U):
                load_idx(e, q, q)
            for b in range(NBUF):
                wait_idx(b)
                start_gather(b, b % NBUF)

            def body(j, carry):
                for b in range(U):
                    cidx = U * j + b
                    buf = b % NBUF
                    wait_gather(buf)
                    scatter(b, buf)
                    count(b)
                    load_idx(e, cidx + U, b)
                    wait_idx((b + NBUF) % U)
                    start_gather((b + NBUF) % U, buf)
                return carry

            lax.fori_loop(0, (CPT - U) // U, body, 0)

            for b in range(U):
                buf = b % NBUF
                wait_gather(buf)
                scatter(b, buf)
                count(b)
                if b < NBUF:
                    wait_idx((b + NBUF) % U)
                    start_gather((b + NBUF) % U, buf)

            plsc.subcore_barrier()
            pltpu.sync_copy(sums_sh.at[pl.ds(s * RPT, RPT)],
                            out_hbm.at[c, e, pl.ds(s * RPT, RPT)])

            @pl.when(c == 0)
            def _():
                pltpu.sync_copy(hist_v, cnt_hbm.at[e, s])

            if e < ETYPES - 1:
                zero_own_slice()

                @pl.when(c == 0)
                def _():
                    zero_hist()

    return k(xh, idx)


_BN = 1024
_BP = _BN // 2
_GRID = NPAD // _BN


def _packed_rcp(cnt_blk):
    rcp = 1.0 / jnp.maximum(cnt_blk, 1.0)
    rcp = rcp.reshape(ETYPES, _BP, 2)
    even = jnp.broadcast_to(rcp[:, :, 0][:, :, None], (ETYPES, _BP, FH))
    odd = jnp.broadcast_to(rcp[:, :, 1][:, :, None], (ETYPES, _BP, FH))
    return jnp.concatenate([even, odd], axis=2)


def _attn_weights_kernel(blk_ref, cnt_ref, w1_ref, b1_ref, lnw_ref, lnb_ref,
                         w2_ref, b2_ref, w_ref, acc_ref):
    i = pl.program_id(0)

    @pl.when(i == 0)
    def _():
        acc_ref[...] = jnp.zeros_like(acc_ref)

    blk = blk_ref[...]
    rcp2 = _packed_rcp(cnt_ref[...].sum(axis=1))
    valid = (lax.broadcasted_iota(jnp.int32, (1, _BP, 1), 1)
             + i * _BP) < N // 2
    rcp2 = jnp.where(valid, rcp2, 0.0)
    for cc in range(NC):
        acc_ref[cc] += (blk[cc] * rcp2).sum(axis=1)

    @pl.when(i == pl.num_programs(0) - 1)
    def _():
        acc = acc_ref[...]
        g = jnp.concatenate(
            [acc[0, :, :FH] + acc[0, :, FH:],
             acc[1, :, :FH] + acc[1, :, FH:]], axis=1) * (1.0 / N)
        h = g @ w1_ref[...] + b1_ref[...]
        mu = jnp.mean(h, axis=-1, keepdims=True)
        var = jnp.mean((h - mu) ** 2, axis=-1, keepdims=True)
        h = (h - mu) * lax.rsqrt(var + 1e-5) * lnw_ref[...] + lnb_ref[...]
        h = jnp.maximum(h, 0.0)
        scores = h @ w2_ref[...] + b2_ref[...]
        eidx = lax.broadcasted_iota(jnp.int32, (ETYPES, 1), 0)
        scores = scores + jnp.where(eidx == ETYPES - 1, _ATT_BIAS[-1],
                                    _ATT_BIAS[0])
        w = jax.nn.sigmoid(scores * 0.5) * 2.0
        w_ref[...] = jnp.clip(w, 0.05, 2.0)


def _combine_kernel(blk_ref, cnt_ref, w_ref, out_ref):
    blk = blk_ref[...]
    rcp2 = _packed_rcp(cnt_ref[...].sum(axis=1))
    w = w_ref[...][:, 0][:, None, None]
    wr = rcp2 * w
    for cc in range(NC):
        out_ref[cc] = (blk[cc] * wr).sum(axis=0)


def _tc_finish(psums, cnt, W1, b1, ln_w, ln_b, W2, b2):
    pview = psums.reshape(NC, ETYPES, NPAD // 2, 128)
    blk_spec = pl.BlockSpec((NC, ETYPES, _BP, 128), lambda i: (0, 0, i, 0))
    cnt_spec = pl.BlockSpec((ETYPES, NS, _BN), lambda i: (0, 0, i))
    full = lambda shape: pl.BlockSpec(shape, lambda i: (0,) * len(shape))
    w = pl.pallas_call(
        _attn_weights_kernel,
        grid=(_GRID,),
        in_specs=[blk_spec, cnt_spec, full((H, H // 4)), full((1, H // 4)),
                  full((1, H // 4)), full((1, H // 4)), full((H // 4, 1)),
                  full((1, 1))],
        out_specs=full((ETYPES, 1)),
        out_shape=jax.ShapeDtypeStruct((ETYPES, 1), jnp.float32),
        scratch_shapes=[pltpu.VMEM((NC, ETYPES, H), jnp.float32)],
    )(pview, cnt, W1, b1.reshape(1, -1), ln_w.reshape(1, -1),
      ln_b.reshape(1, -1), W2, b2.reshape(1, -1))
    fp = pl.pallas_call(
        _combine_kernel,
        grid=(_GRID,),
        in_specs=[blk_spec, cnt_spec, full((ETYPES, 1))],
        out_specs=pl.BlockSpec((NC, _BP, 128), lambda i: (0, i, 0)),
        out_shape=jax.ShapeDtypeStruct((NC, NPAD // 2, 128), jnp.float32),
    )(pview, cnt, w)
    return jnp.concatenate(
        [fp[0, :N // 2].reshape(N, FH), fp[1, :N // 2].reshape(N, FH)],
        axis=1)


def kernel(x, ei_seq, ei_str_knn, ei_str_dis, ei_surf, ei_lrr,
           W1, b1, ln_w, ln_b, W2, b2):
    xh = jnp.stack([x[:, :FH], x[:, FH:]])
    xh = jnp.pad(xh, ((0, 0), (0, NPAD - N), (0, 0)))
    idxs = []
    pad = EPAD - E
    for ei in (ei_seq, ei_str_knn, ei_str_dis, ei_surf, ei_lrr):
        src = jnp.concatenate(
            [ei[0], jnp.zeros((pad,), jnp.int32)]).reshape(NS, CPT, 1, K)
        dst = jnp.concatenate(
            [ei[1], jnp.full((pad,), DUMMY, jnp.int32)]).reshape(NS, CPT, 1, K)
        idxs.append(jnp.concatenate([src, dst], axis=2))
    psums, cnt = _sc_agg(xh, jnp.stack(idxs))
    return _tc_finish(psums, cnt, W1, b1, ln_w, ln_b, W2, b2)


# speedup vs baseline: 1.0222x; 1.0222x over previous
import functools

import jax
import jax.numpy as jnp
from jax import lax
from jax.experimental import pallas as pl
from jax.experimental.pallas import tpu as pltpu
from jax.experimental.pallas import tpu_sc as plsc

N = 10000
H = 128
E = 320000
ETYPES = 5
_ATT_BIAS = (-4.0, -4.0, -4.0, -4.0, -2.772)

NC, NS, L = 2, 16, 16
FH = H // NC
K = 128
NBUF = 2
U = 2 * NBUF
CPT = 160
EPAD = NS * CPT * K
NPAD = 10240
RPT = NPAD // NS
DUMMY = N


def _sc_agg(xh, idx):
    mesh = plsc.VectorSubcoreMesh(core_axis_name="c", subcore_axis_name="s")

    @functools.partial(
        pl.kernel,
        out_type=(
            jax.ShapeDtypeStruct((NC, ETYPES, NPAD, FH), jnp.float32),
            jax.ShapeDtypeStruct((ETYPES, NS, NPAD), jnp.float32),
        ),
        mesh=mesh,
        scratch_types=[
            pltpu.VMEM((U, 2, K), jnp.int32),
            pltpu.VMEM((NBUF, K, FH), jnp.float32),
            pltpu.VMEM((NPAD,), jnp.float32),
            pltpu.VMEM_SHARED((NPAD, FH), jnp.float32),
            pltpu.VMEM_SHARED((NPAD, FH), jnp.float32),
            pltpu.SemaphoreType.DMA((U,)),
            pltpu.SemaphoreType.DMA((NBUF,)),
        ],
        compiler_params=pltpu.CompilerParams(use_tc_tiling_on_sc=False,
                                             needs_layout_passes=False),
    )
    def k(xh_hbm, idx_hbm, out_hbm, cnt_hbm, idx_v, rows_v, hist_v, x_sh,
          sums_sh, isem, gsem):
        c = lax.axis_index("c")
        s = lax.axis_index("s")
        zvec = jnp.zeros((L,), jnp.float32)
        ones = jnp.ones((L,), jnp.float32)

        def zrow(i, carry):
            for j in range(FH // L):
                rows_v[0, i, pl.ds(j * L, L)] = zvec
            return carry

        def zero_own_slice():
            lax.fori_loop(0, K, zrow, 0)
            for q in range(RPT // K):
                pltpu.sync_copy(rows_v.at[0],
                                sums_sh.at[pl.ds(s * RPT + q * K, K)])

        def zero_hist():
            def zh(i, carry):
                hist_v[pl.ds(i * L, L)] = zvec
                return carry
            lax.fori_loop(0, NPAD // L, zh, 0)

        def load_idx(e, chunk, q):
            pltpu.async_copy(idx_hbm.at[e, s, chunk], idx_v.at[q],
                             isem.at[q])

        def wait_idx(q):
            pltpu.make_async_copy(idx_hbm.at[0, 0, 0], idx_v.at[q],
                                  isem.at[q]).wait()

        def start_gather(q, b):
            pltpu.async_copy(x_sh.at[idx_v.at[q, 0]], rows_v.at[b],
                             gsem.at[b])

        def wait_gather(b):
            pltpu.make_async_copy(xh_hbm.at[0, pl.ds(0, K)], rows_v.at[b],
                                  gsem.at[b]).wait()

        def scatter(q, b):
            pltpu.sync_copy(rows_v.at[b], sums_sh.at[idx_v.at[q, 1]],
                            add=True)

        def count(q):
            @pl.when(c == 0)
            def _():
                for j in range(K // L):
                    dstv = idx_v[q, 1, pl.ds(j * L, L)]
                    plsc.addupdate_scatter(hist_v, [dstv], ones)

        pltpu.sync_copy(xh_hbm.at[c, pl.ds(s * RPT, RPT)],
                        x_sh.at[pl.ds(s * RPT, RPT)])
        zero_hist()
        zero_own_slice()

        for e in range(ETYPES):
            plsc.subcore_barrier()
            for q in range(U):
                load_idx(e, q, q)
            for b in range(NBUF):
                wait_idx(b)
                start_gather(b, b % NBUF)

            def body(j, carry):
                for b in range(U):
                    cidx = U * j + b
                    buf = b % NBUF
                    wait_gather(buf)
                    scatter(b, buf)
                    count(b)
                    load_idx(e, cidx + U, b)
                    wait_idx((b + NBUF) % U)
                    start_gather((b + NBUF) % U, buf)
                return carry

            lax.fori_loop(0, (CPT - U) // U, body, 0)

            for b in range(U):
                buf = b % NBUF
                wait_gather(buf)
                scatter(b, buf)
                count(b)
                if b < NBUF:
                    wait_idx((b + NBUF) % U)
                    start_gather((b + NBUF) % U, buf)

            plsc.subcore_barrier()
            pltpu.sync_copy(sums_sh.at[pl.ds(s * RPT, RPT)],
                            out_hbm.at[c, e, pl.ds(s * RPT, RPT)])

            @pl.when(c == 0)
            def _():
                pltpu.sync_copy(hist_v, cnt_hbm.at[e, s])

            if e < ETYPES - 1:
                zero_own_slice()

                @pl.when(c == 0)
                def _():
                    zero_hist()

    return k(xh, idx)


_BN = 1024
_BP = _BN // 2
_GRID = NPAD // _BN


def _packed_rcp(cnt_blk):
    rcp = 1.0 / jnp.maximum(cnt_blk, 1.0)
    rcp = rcp.reshape(ETYPES, _BP, 2)
    even = jnp.broadcast_to(rcp[:, :, 0][:, :, None], (ETYPES, _BP, FH))
    odd = jnp.broadcast_to(rcp[:, :, 1][:, :, None], (ETYPES, _BP, FH))
    return jnp.concatenate([even, odd], axis=2)


def _attn_weights_kernel(blk_ref, cnt_ref, w1_ref, b1_ref, lnw_ref, lnb_ref,
                         w2_ref, b2_ref, w_ref, acc_ref):
    i = pl.program_id(0)

    @pl.when(i == 0)
    def _():
        acc_ref[...] = jnp.zeros_like(acc_ref)

    blk = blk_ref[...]
    rcp2 = _packed_rcp(cnt_ref[...].sum(axis=1))
    valid = (lax.broadcasted_iota(jnp.int32, (1, _BP, 1), 1)
             + i * _BP) < N // 2
    rcp2 = jnp.where(valid, rcp2, 0.0)
    for cc in range(NC):
        acc_ref[cc] += (blk[cc] * rcp2).sum(axis=1)

    @pl.when(i == pl.num_programs(0) - 1)
    def _():
        acc = acc_ref[...]
        g = jnp.concatenate(
            [acc[0, :, :FH] + acc[0, :, FH:],
             acc[1, :, :FH] + acc[1, :, FH:]], axis=1) * (1.0 / N)
        h = g @ w1_ref[...] + b1_ref[...]
        mu = jnp.mean(h, axis=-1, keepdims=True)
        var = jnp.mean((h - mu) ** 2, axis=-1, keepdims=True)
        h = (h - mu) * lax.rsqrt(var + 1e-5) * lnw_ref[...] + lnb_ref[...]
        h = jnp.maximum(h, 0.0)
        scores = h @ w2_ref[...] + b2_ref[...]
        eidx = lax.broadcasted_iota(jnp.int32, (ETYPES, 1), 0)
        scores = scores + jnp.where(eidx == ETYPES - 1, _ATT_BIAS[-1],
                                    _ATT_BIAS[0])
        w = jax.nn.sigmoid(scores * 0.5) * 2.0
        w_ref[...] = jnp.clip(w, 0.05, 2.0)


def _combine_kernel(blk_ref, cnt_ref, w_ref, out_ref):
    blk = blk_ref[...]
    rcp2 = _packed_rcp(cnt_ref[...].sum(axis=1))
    w = w_ref[...][:, 0][:, None, None]
    wr = rcp2 * w
    for cc in range(NC):
        out_ref[cc] = (blk[cc] * wr).sum(axis=0)


def _tc_finish(psums, cnt, W1, b1, ln_w, ln_b, W2, b2):
    pview = psums.reshape(NC, ETYPES, NPAD // 2, 128)
    blk_spec = pl.BlockSpec((NC, ETYPES, _BP, 128), lambda i: (0, 0, i, 0))
    cnt_spec = pl.BlockSpec((ETYPES, NS, _BN), lambda i: (0, 0, i))
    full = lambda shape: pl.BlockSpec(shape, lambda i: (0,) * len(shape))
    w = pl.pallas_call(
        _attn_weights_kernel,
        grid=(_GRID,),
        in_specs=[blk_spec, cnt_spec, full((H, H // 4)), full((1, H // 4)),
                  full((1, H // 4)), full((1, H // 4)), full((H // 4, 1)),
                  full((1, 1))],
        out_specs=full((ETYPES, 1)),
        out_shape=jax.ShapeDtypeStruct((ETYPES, 1), jnp.float32),
        scratch_shapes=[pltpu.VMEM((NC, ETYPES, H), jnp.float32)],
    )(pview, cnt, W1, b1.reshape(1, -1), ln_w.reshape(1, -1),
      ln_b.reshape(1, -1), W2, b2.reshape(1, -1))
    fp = pl.pallas_call(
        _combine_kernel,
        grid=(_GRID,),
        in_specs=[blk_spec, cnt_spec, full((ETYPES, 1))],
        out_specs=pl.BlockSpec((NC, _BP, 128), lambda i: (0, i, 0)),
        out_shape=jax.ShapeDtypeStruct((NC, NPAD // 2, 128), jnp.float32),
    )(pview, cnt, w)
    return jnp.concatenate(
        [fp[0, :N // 2].reshape(N, FH), fp[1, :N // 2].reshape(N, FH)],
        axis=1)


def kernel(x, ei_seq, ei_str_knn, ei_str_dis, ei_surf, ei_lrr,
           W1, b1, ln_w, ln_b, W2, b2):
    xh = jnp.stack([x[:, :FH], x[:, FH:]])
    xh = jnp.pad(xh, ((0, 0), (0, NPAD - N), (0, 0)))
    idxs = []
    pad = EPAD - E
    for ei in (ei_seq, ei_str_knn, ei_str_dis, ei_surf, ei_lrr):
        src = jnp.concatenate(
            [ei[0], jnp.zeros((pad,), jnp.int32)]).reshape(NS, CPT, 1, K)
        dst = jnp.concatenate(
            [ei[1], jnp.full((pad,), DUMMY, jnp.int32)]).reshape(NS, CPT, 1, K)
        idxs.append(jnp.concatenate([src, dst], axis=2))
    psums, cnt = _sc_agg(xh, jnp.stack(idxs))
    return _tc_finish(psums, cnt, W1, b1, ln_w, ln_b, W2, b2)
